# baseline (speedup 1.0000x reference)
import jax
import jax.numpy as jnp
from jax import lax
from jax.experimental import pallas as pl
from jax.experimental.pallas import tpu as pltpu
from jax.experimental.pallas import tpu_sc as plsc

B = 16384
D = 16
NF = 26
NC = 2
NS = 16
NW = NC * NS
BPW = B // NW
NR = 16


def _body(*refs):
    idx_refs = refs[:NF]
    tbl_refs = refs[NF:2 * NF]
    out_refs = refs[2 * NF:3 * NF]
    rest = refs[3 * NF:]
    idx_v = rest[0]
    grpbufs = rest[1:1 + NR]
    xbufs = rest[1 + NR:3 + NR]
    sem_g = rest[3 + NR:3 + 2 * NR]
    sem_w = rest[3 + 2 * NR:5 + 2 * NR]

    wid = lax.axis_index("s") * NC + lax.axis_index("c")
    base = wid * BPW
    iota = jax.lax.iota(jnp.int32, 16)
    zero = iota * 0

    writes = [None] * NF
    for f in range(NF):
        xbuf = xbufs[f % 2]
        if f >= 2:
            writes[f - 2].wait()
        pltpu.sync_copy(idx_refs[f].at[pl.ds(base, BPW)], idx_v)

        ivec0 = idx_v[pl.ds(0, NR)]
        copies = [None] * NR
        for s in range(NR):
            g8 = pl.multiple_of(ivec0[s] & ~jnp.int32(7), 8)
            copies[s] = pltpu.async_copy(
                tbl_refs[f].at[pl.ds(g8, 8), :], grpbufs[s], sem_g[s])

        def block(blk, _, f=f, copies=copies, xbuf=xbuf):
            ivec = idx_v[pl.ds(blk * NR, NR)]
            nvec = idx_v[pl.ds(jnp.minimum(blk * NR + NR, BPW - NR), NR)]
            for s in range(NR):
                j = blk * NR + s
                copies[s].wait()
                r = ivec[s] & jnp.int32(7)
                row = grpbufs[s][r, pl.ds(0, D)]
                plsc.store_scatter(xbuf, [iota, zero + j], row)

                @pl.when(blk * NR + NR < BPW)
                def _():
                    g8 = pl.multiple_of(nvec[s] & ~jnp.int32(7), 8)
                    pltpu.async_copy(
                        tbl_refs[f].at[pl.ds(g8, 8), :], grpbufs[s], sem_g[s])
            return ()

        lax.fori_loop(0, BPW // NR, block, (), unroll=False)

        writes[f] = pltpu.async_copy(
            xbuf, out_refs[f].at[:, pl.ds(base, BPW)], sem_w[f % 2])
    writes[NF - 2].wait()
    writes[NF - 1].wait()


def kernel(f0, f1, f2, f3, f4, f5, f6, f7, f8, f9, f10, f11, f12, f13, f14,
           f15, f16, f17, f18, f19, f20, f21, f22, f23, f24, f25,
           W_f0, W_f1, W_f2, W_f3, W_f4, W_f5, W_f6, W_f7, W_f8, W_f9,
           W_f10, W_f11, W_f12, W_f13, W_f14, W_f15, W_f16, W_f17, W_f18,
           W_f19, W_f20, W_f21, W_f22, W_f23, W_f24, W_f25):
    idx = (f0, f1, f2, f3, f4, f5, f6, f7, f8, f9, f10, f11, f12, f13, f14,
           f15, f16, f17, f18, f19, f20, f21, f22, f23, f24, f25)
    tbls = (W_f0, W_f1, W_f2, W_f3, W_f4, W_f5, W_f6, W_f7, W_f8, W_f9,
            W_f10, W_f11, W_f12, W_f13, W_f14, W_f15, W_f16, W_f17, W_f18,
            W_f19, W_f20, W_f21, W_f22, W_f23, W_f24, W_f25)

    mesh = plsc.VectorSubcoreMesh(core_axis_name="c", subcore_axis_name="s",
                                  num_cores=NC, num_subcores=NS)
    run = pl.kernel(
        _body,
        out_type=tuple(jax.ShapeDtypeStruct((D, B), jnp.float32)
                       for _ in range(NF)),
        mesh=mesh,
        scratch_types=(
            [pltpu.VMEM((BPW,), jnp.int32)]
            + [pltpu.VMEM((8, D), jnp.float32) for _ in range(NR)]
            + [pltpu.VMEM((D, BPW), jnp.float32) for _ in range(2)]
            + [pltpu.SemaphoreType.DMA for _ in range(NR)]
            + [pltpu.SemaphoreType.DMA for _ in range(2)]
        ),
        compiler_params=pltpu.CompilerParams(needs_layout_passes=False),
    )
    outs = run(*idx, *tbls)
    return jnp.concatenate([o.T for o in outs], axis=-1)


# speedup vs baseline: 1.1883x; 1.1883x over previous
import jax
import jax.numpy as jnp
from jax import lax
from jax.experimental import pallas as pl
from jax.experimental.pallas import tpu as pltpu
from jax.experimental.pallas import tpu_sc as plsc

B = 16384
D = 16
NF = 26
NC = 2
NS = 16
NW = NC * NS
BPW = B // NW
CH = 512
NCHF = BPW // CH


def _body(*refs):
    idx_refs = refs[:NF]
    tbl_refs = refs[NF:2 * NF]
    out_refs = refs[2 * NF:3 * NF]
    rest = refs[3 * NF:]
    idx_v = rest[0]
    gidx = rest[1:3]
    padbufs = [rest[3], rest[3]]
    xbufs = rest[4:6]
    sem_i = rest[6]
    sem_g = rest[7:9]
    sem_w = rest[9:11]

    wid = lax.axis_index("s") * NC + lax.axis_index("c")
    base = wid * BPW
    iota = jax.lax.iota(jnp.int32, 16)

    idx_copies = [
        pltpu.async_copy(idx_refs[f].at[pl.ds(base, BPW)],
                         idx_v.at[pl.ds(f * BPW, BPW)], sem_i)
        for f in range(NF)
    ]
    for cpy in idx_copies:
        cpy.wait()

    writes = [None] * (NF * NCHF)
    step = 0
    for f in range(NF):
        for ch in range(NCHF):
            p = step % 2
            off = f * BPW + ch * CH

            def gshift(jb, _, p=p, off=off):
                ivec = idx_v[pl.ds(off + jb * 16, 16)]
                gidx[p][pl.ds(jb * 16, 16)] = \
                    jax.lax.shift_right_logical(ivec, 3)
                return ()

            lax.fori_loop(0, CH // 16, gshift, (), unroll=False)

            if step >= 2:
                writes[step - 2].wait()
            pltpu.async_copy(tbl_refs[f].at[gidx[p]], padbufs[p],
                             sem_g[p]).wait()

            def extract(jb, _, p=p, off=off):
                jvec = iota + jb * 16
                ivec = idx_v[pl.ds(off + jb * 16, 16)]
                rvec = (ivec & jnp.int32(7)) * 16
                for l in range(D):
                    col = plsc.load_gather(padbufs[p], [jvec, rvec + l])
                    xbufs[p][l, pl.ds(jb * 16, 16)] = col
                return ()

            lax.fori_loop(0, CH // 16, extract, (), unroll=False)

            writes[step] = pltpu.async_copy(
                xbufs[p],
                out_refs[f].at[:, pl.ds(base + ch * CH, CH)],
                sem_w[p])
            step += 1
    writes[step - 2].wait()
    writes[step - 1].wait()


def kernel(f0, f1, f2, f3, f4, f5, f6, f7, f8, f9, f10, f11, f12, f13, f14,
           f15, f16, f17, f18, f19, f20, f21, f22, f23, f24, f25,
           W_f0, W_f1, W_f2, W_f3, W_f4, W_f5, W_f6, W_f7, W_f8, W_f9,
           W_f10, W_f11, W_f12, W_f13, W_f14, W_f15, W_f16, W_f17, W_f18,
           W_f19, W_f20, W_f21, W_f22, W_f23, W_f24, W_f25):
    idx = (f0, f1, f2, f3, f4, f5, f6, f7, f8, f9, f10, f11, f12, f13, f14,
           f15, f16, f17, f18, f19, f20, f21, f22, f23, f24, f25)
    tbls = tuple(
        w.reshape(w.shape[0] // 8, 128)
        for w in (W_f0, W_f1, W_f2, W_f3, W_f4, W_f5, W_f6, W_f7, W_f8,
                  W_f9, W_f10, W_f11, W_f12, W_f13, W_f14, W_f15, W_f16,
                  W_f17, W_f18, W_f19, W_f20, W_f21, W_f22, W_f23, W_f24,
                  W_f25))

    mesh = plsc.VectorSubcoreMesh(core_axis_name="c", subcore_axis_name="s",
                                  num_cores=NC, num_subcores=NS)
    run = pl.kernel(
        _body,
        out_type=tuple(jax.ShapeDtypeStruct((D, B), jnp.float32)
                       for _ in range(NF)),
        mesh=mesh,
        scratch_types=(
            [pltpu.VMEM((NF * BPW,), jnp.int32)]
            + [pltpu.VMEM((CH,), jnp.int32) for _ in range(2)]
            + [pltpu.VMEM((CH, 128), jnp.float32)]
            + [pltpu.VMEM((D, CH), jnp.float32) for _ in range(2)]
            + [pltpu.SemaphoreType.DMA]
            + [pltpu.SemaphoreType.DMA for _ in range(2)]
            + [pltpu.SemaphoreType.DMA for _ in range(2)]
        ),
        compiler_params=pltpu.CompilerParams(needs_layout_passes=False),
    )
    outs = run(*idx, *tbls)
    return jnp.concatenate([o.T for o in outs], axis=-1)


# speedup vs baseline: 1.2736x; 1.0717x over previous
import jax
import jax.numpy as jnp
from jax import lax
from jax.experimental import pallas as pl
from jax.experimental.pallas import tpu as pltpu
from jax.experimental.pallas import tpu_sc as plsc

B = 16384
D = 16
NF = 26
NBIG = 16
CSMALL = 1000
NC = 2
NS = 16
NW = NC * NS
BPW = B // NW
NBUF = 8


def _body(*refs):
    idx_ref = refs[0]
    tbl_refs = refs[1:1 + NBIG + 1]
    out_ref = refs[NBIG + 2]
    rest = refs[NBIG + 3:]
    idx_v = rest[0]
    bufs = rest[1:1 + NBUF]
    sem_i = rest[1 + NBUF]
    sem_g = rest[2 + NBUF:2 + 2 * NBUF]
    sem_w = rest[2 + 2 * NBUF:2 + 3 * NBUF]

    wid = lax.axis_index("s") * NC + lax.axis_index("c")
    base = wid * BPW

    idx_copies = [
        pltpu.async_copy(idx_ref.at[pl.ds(f * B + base, BPW)],
                         idx_v.at[pl.ds(f * BPW, BPW)], sem_i)
        for f in range(NF)
    ]
    for cpy in idx_copies:
        cpy.wait()

    def tbl(f):
        return tbl_refs[f] if f < NBIG else tbl_refs[NBIG]

    gathers = [None] * NF
    writes = [None] * NF
    for f in range(NBUF):
        gathers[f] = pltpu.async_copy(
            tbl(f).at[idx_v.at[pl.ds(f * BPW, BPW)]], bufs[f], sem_g[f])
    for f in range(NF):
        slot = f % NBUF
        gathers[f].wait()
        writes[f] = pltpu.async_copy(
            bufs[slot],
            out_ref.at[pl.ds(base, BPW), pl.ds(f * D, D)],
            sem_w[slot])
        g = f + NBUF
        if g < NF:
            writes[f].wait()
            gathers[g] = pltpu.async_copy(
                tbl(g).at[idx_v.at[pl.ds(g * BPW, BPW)]], bufs[slot],
                sem_g[slot])
    for f in range(NF - NBUF, NF):
        writes[f].wait()


def kernel(f0, f1, f2, f3, f4, f5, f6, f7, f8, f9, f10, f11, f12, f13, f14,
           f15, f16, f17, f18, f19, f20, f21, f22, f23, f24, f25,
           W_f0, W_f1, W_f2, W_f3, W_f4, W_f5, W_f6, W_f7, W_f8, W_f9,
           W_f10, W_f11, W_f12, W_f13, W_f14, W_f15, W_f16, W_f17, W_f18,
           W_f19, W_f20, W_f21, W_f22, W_f23, W_f24, W_f25):
    idx = [f0, f1, f2, f3, f4, f5, f6, f7, f8, f9, f10, f11, f12, f13, f14,
           f15, f16, f17, f18, f19, f20, f21, f22, f23, f24, f25]
    for i in range(NBIG, NF):
        idx[i] = idx[i] + (i - NBIG) * CSMALL
    idx_all = jnp.stack(idx).reshape(NF * B)

    big_tbls = (W_f0, W_f1, W_f2, W_f3, W_f4, W_f5, W_f6, W_f7, W_f8, W_f9,
                W_f10, W_f11, W_f12, W_f13, W_f14, W_f15)
    small = jnp.concatenate(
        (W_f16, W_f17, W_f18, W_f19, W_f20, W_f21, W_f22, W_f23, W_f24,
         W_f25), axis=0)

    mesh = plsc.VectorSubcoreMesh(core_axis_name="c", subcore_axis_name="s",
                                  num_cores=NC, num_subcores=NS)
    run = pl.kernel(
        _body,
        out_type=jax.ShapeDtypeStruct((B, NF * D), jnp.float32),
        mesh=mesh,
        scratch_types=(
            [pltpu.VMEM((NF * BPW,), jnp.int32)]
            + [pltpu.VMEM((BPW, D), jnp.float32) for _ in range(NBUF)]
            + [pltpu.SemaphoreType.DMA]
            + [pltpu.SemaphoreType.DMA for _ in range(2 * NBUF)]
        ),
        compiler_params=pltpu.CompilerParams(use_tc_tiling_on_sc=False),
    )
    return run(idx_all, *big_tbls, small)
